# Initial kernel scaffold; baseline (speedup 1.0000x reference)
import jax
import jax.numpy as jnp
from jax.experimental import pallas as pl


def kernel(X, E_real, E_img, R_real, R_img, T_emb):
    raise NotImplementedError("write your pallas kernel here")


# speedup vs baseline: 1.8643x; 1.8643x over previous
import functools

import jax
import jax.numpy as jnp
from jax import lax
from jax.experimental import pallas as pl
from jax.experimental.pallas import tpu as pltpu
from jax.experimental.pallas import tpu_sc as plsc

B = 16384
DIM = 32
HALF = 16

NUM_CORES = 2
NUM_SUBCORES = 16
NUM_WORKERS = NUM_CORES * NUM_SUBCORES
PER_W = B // NUM_WORKERS
CHUNK = 128
NCHUNK = PER_W // CHUNK
BLK = 16
NBLK = CHUNK // BLK


def _sincos_body(t_ref, cs_ref):
    t = t_ref[...]
    cs_ref[...] = jnp.concatenate([jnp.cos(t), jnp.sin(t)], axis=1)


def _sincos_table(T_emb):
    n, d = T_emb.shape
    return pl.pallas_call(
        _sincos_body,
        out_shape=jax.ShapeDtypeStruct((n, 2 * d), jnp.float32),
    )(T_emb)


def _sc_body(h_hbm, t_hbm, r_hbm, d_hbm, er_hbm, ei_hbm, rr_hbm, ri_hbm,
             cs_hbm, out_hbm,
             h_v, t_v, r_v, d_v, hre, him, tre, tim, rre, rim, csb,
             out_v, sem):
    wid = lax.axis_index("s") * NUM_CORES + lax.axis_index("c")
    base = wid * PER_W

    iota = lax.iota(jnp.int32, HALF)

    for k in range(NCHUNK):
        off = base + k * CHUNK
        pltpu.sync_copy(h_hbm.at[pl.ds(off, CHUNK)], h_v)
        pltpu.sync_copy(t_hbm.at[pl.ds(off, CHUNK)], t_v)
        pltpu.sync_copy(r_hbm.at[pl.ds(off, CHUNK)], r_v)
        pltpu.sync_copy(d_hbm.at[pl.ds(off, CHUNK)], d_v)
        copies = [
            pltpu.async_copy(er_hbm.at[h_v], hre, sem),
            pltpu.async_copy(ei_hbm.at[h_v], him, sem),
            pltpu.async_copy(er_hbm.at[t_v], tre, sem),
            pltpu.async_copy(ei_hbm.at[t_v], tim, sem),
            pltpu.async_copy(rr_hbm.at[r_v], rre, sem),
            pltpu.async_copy(ri_hbm.at[r_v], rim, sem),
            pltpu.async_copy(cs_hbm.at[d_v], csb, sem),
        ]
        for cp in copies:
            cp.wait()

        def blk_body(b, _, k=k):
            el0 = b * BLK
            acc = jnp.zeros((HALF,), jnp.float32)
            for e in range(BLK):
                el = el0 + e
                hr0 = hre[el, pl.ds(0, HALF)]
                hr1 = hre[el, pl.ds(HALF, HALF)]
                hi0 = him[el, pl.ds(0, HALF)]
                hi1 = him[el, pl.ds(HALF, HALF)]
                tr0 = tre[el, pl.ds(0, HALF)]
                tr1 = tre[el, pl.ds(HALF, HALF)]
                ti0 = tim[el, pl.ds(0, HALF)]
                ti1 = tim[el, pl.ds(HALF, HALF)]
                rr0 = rre[el, pl.ds(0, HALF)]
                rr1 = rre[el, pl.ds(HALF, HALF)]
                ri0 = rim[el, pl.ds(0, HALF)]
                ri1 = rim[el, pl.ds(HALF, HALF)]
                c0 = csb[el, pl.ds(0, HALF)]
                c1 = csb[el, pl.ds(HALF, HALF)]
                s0 = csb[el, pl.ds(2 * HALF, HALF)]
                s1 = csb[el, pl.ds(3 * HALF, HALF)]
                p0 = (jnp.abs(hr0 * c0 - hi0 * s0 + rr0 - (tr0 * c0 - ti0 * s0))
                      + jnp.abs(hr0 * s0 + hi0 * c0 + ri0 + (tr0 * s0 + ti0 * c0)))
                p1 = (jnp.abs(hr1 * c1 - hi1 * s1 + rr1 - (tr1 * c1 - ti1 * s1))
                      + jnp.abs(hr1 * s1 + hi1 * c1 + ri1 + (tr1 * s1 + ti1 * c1)))
                acc = jnp.where(iota == e, jnp.sum(p0 + p1), acc)
            out_v[pl.ds(k * CHUNK + el0, BLK)] = acc
            return 0

        lax.fori_loop(0, NBLK, blk_body, 0)

    pltpu.sync_copy(out_v, out_hbm.at[pl.ds(base, PER_W)])


@functools.partial(
    pl.kernel,
    out_type=jax.ShapeDtypeStruct((B,), jnp.float32),
    mesh=plsc.VectorSubcoreMesh(core_axis_name="c", subcore_axis_name="s"),
    compiler_params=pltpu.CompilerParams(
        needs_layout_passes=False, use_tc_tiling_on_sc=False),
    scratch_types=[
        pltpu.VMEM((CHUNK,), jnp.int32),
        pltpu.VMEM((CHUNK,), jnp.int32),
        pltpu.VMEM((CHUNK,), jnp.int32),
        pltpu.VMEM((CHUNK,), jnp.int32),
        pltpu.VMEM((CHUNK, DIM), jnp.float32),
        pltpu.VMEM((CHUNK, DIM), jnp.float32),
        pltpu.VMEM((CHUNK, DIM), jnp.float32),
        pltpu.VMEM((CHUNK, DIM), jnp.float32),
        pltpu.VMEM((CHUNK, DIM), jnp.float32),
        pltpu.VMEM((CHUNK, DIM), jnp.float32),
        pltpu.VMEM((CHUNK, 2 * DIM), jnp.float32),
        pltpu.VMEM((PER_W,), jnp.float32),
        pltpu.SemaphoreType.DMA,
    ],
)
def _sc_lookup(h_hbm, t_hbm, r_hbm, d_hbm, er_hbm, ei_hbm, rr_hbm, ri_hbm,
               cs_hbm, out_hbm, *scratch):
    _sc_body(h_hbm, t_hbm, r_hbm, d_hbm, er_hbm, ei_hbm, rr_hbm, ri_hbm,
             cs_hbm, out_hbm, *scratch)


def kernel(X, E_real, E_img, R_real, R_img, T_emb):
    h = X[:, 0]
    t = X[:, 1]
    r = X[:, 2]
    d = X[:, 3]
    cs = _sincos_table(T_emb)
    return _sc_lookup(h, t, r, d, E_real, E_img, R_real, R_img, cs)
